# Initial kernel scaffold; baseline (speedup 1.0000x reference)
import jax
import jax.numpy as jnp
from jax.experimental import pallas as pl


def kernel(spectrogram):
    raise NotImplementedError("write your pallas kernel here")


# speedup vs baseline: 42.1900x; 42.1900x over previous
import functools

import numpy as np
import jax
import jax.numpy as jnp
from jax import lax
from jax.experimental import pallas as pl
from jax.experimental.pallas import tpu as pltpu
from jax.experimental.pallas import tpu_sc as plsc

_SAMPLE_RATE = 22050
_N_FFT = 2048
_FMIN = 200.0
_N_BANDS = 6
_QUANTILE = 0.02


def _band_ranges():
    freq = np.linspace(0.0, _SAMPLE_RATE * 0.5, 1 + _N_FFT // 2)
    octa = np.zeros(_N_BANDS + 2)
    octa[1:] = _FMIN * 2.0 ** np.arange(0, _N_BANDS + 1, dtype=np.float32)
    bands = []
    for k in range(_N_BANDS + 1):
        f_low, f_high = octa[k], octa[k + 1]
        mask = (freq >= f_low) & (freq <= f_high)
        idx = np.flatnonzero(mask)
        if k > 0:
            mask[idx[0] - 1] = True
        if k == _N_BANDS:
            mask[idx[-1] + 1:] = True
        sub_idx = np.flatnonzero(mask)
        if k < _N_BANDS:
            sub_idx = sub_idx[:-1]
        q = int(max(np.rint(_QUANTILE * mask.sum()), 1))
        assert np.all(np.diff(sub_idx) == 1)
        bands.append((int(sub_idx[0]), int(len(sub_idx)), q))
    return bands


_BANDS = _band_ranges()
_NB = len(_BANDS)

_B, _F, _T = 16, 1025, 2048
_L = 16
_C = 128
_NCHUNK = _T // _C
_NW = 32
_PER_W = _B * _NCHUNK // _NW

_CHUNKS = ((0, 304, (0, 1, 2, 3, 4)), (296, 304, (5,)), (592, 433, (6,)))
_BUF_ROWS = max(c[1] for c in _CHUNKS)


def _sc_body(spec, peak, valley, buf, pk, vl):
    wid = lax.axis_index("s") * 2 + lax.axis_index("c")

    def item(i, carry):
        it = wid * _PER_W + i
        b = it // _NCHUNK
        t0 = (it % _NCHUNK) * _C
        for alo, plen, bids in _CHUNKS:
            pltpu.sync_copy(spec.at[b, pl.ds(alo, plen), pl.ds(t0, _C)],
                            buf.at[pl.ds(0, plen)])
            for g in range(_C // _L):
                sl = pl.ds(g * _L, _L)
                for bi in bids:
                    lo, ln, q = _BANDS[bi]
                    off = lo - alo
                    inf = jnp.full((_L,), jnp.inf, jnp.float32)
                    init = ((inf,) * q, (-inf,) * q)

                    def row(r, st, off=off, q=q, sl=sl):
                        bot, top = st
                        v = buf[off + r, sl]
                        w = v
                        nb, nt = [], []
                        for j in range(q):
                            nb.append(jnp.minimum(bot[j], v))
                            v = jnp.maximum(bot[j], v)
                        for j in range(q):
                            nt.append(jnp.maximum(top[j], w))
                            w = jnp.minimum(top[j], w)
                        return tuple(nb), tuple(nt)

                    bot, top = lax.fori_loop(0, ln, row, init)
                    sb, st_ = bot[0], top[0]
                    for j in range(1, q):
                        sb = sb + bot[j]
                        st_ = st_ + top[j]
                    r = jnp.float32(1.0 / q)
                    vl[bi, sl] = sb * r
                    pk[bi, sl] = st_ * r
        pltpu.sync_copy(pk, peak.at[b, :, pl.ds(t0, _C)])
        pltpu.sync_copy(vl, valley.at[b, :, pl.ds(t0, _C)])
        return carry

    lax.fori_loop(0, _PER_W, item, 0)


@functools.lru_cache(maxsize=1)
def _sc_extremes():
    return pl.kernel(
        _sc_body,
        out_type=[jax.ShapeDtypeStruct((_B, _NB, _T), jnp.float32),
                  jax.ShapeDtypeStruct((_B, _NB, _T), jnp.float32)],
        mesh=plsc.VectorSubcoreMesh(core_axis_name="c", subcore_axis_name="s",
                                    num_cores=2, num_subcores=16),
        scratch_types=[pltpu.VMEM((_BUF_ROWS, _C), jnp.float32),
                       pltpu.VMEM((_NB, _C), jnp.float32),
                       pltpu.VMEM((_NB, _C), jnp.float32)],
    )


def _db_body(pk_ref, vl_ref, o_ref):
    pk = pk_ref[0]
    vb = vl_ref[0]
    lp = 10.0 * jnp.log10(jnp.maximum(pk, 1e-10))
    lv = 10.0 * jnp.log10(jnp.maximum(vb, 1e-10))
    lp = jnp.maximum(lp, jnp.max(lp) - 80.0)
    lv = jnp.maximum(lv, jnp.max(lv) - 80.0)
    o_ref[0] = lp - lv


def kernel(spectrogram):
    peak, valley = _sc_extremes()(spectrogram)
    return pl.pallas_call(
        _db_body,
        grid=(_B,),
        in_specs=[pl.BlockSpec((1, _NB, _T), lambda b: (b, 0, 0))] * 2,
        out_specs=pl.BlockSpec((1, _NB, _T), lambda b: (b, 0, 0)),
        out_shape=jax.ShapeDtypeStruct((_B, _NB, _T), jnp.float32),
    )(peak, valley)


# speedup vs baseline: 61.2900x; 1.4527x over previous
import functools

import numpy as np
import jax
import jax.numpy as jnp
from jax import lax
from jax.experimental import pallas as pl
from jax.experimental.pallas import tpu as pltpu
from jax.experimental.pallas import tpu_sc as plsc

_SAMPLE_RATE = 22050
_N_FFT = 2048
_FMIN = 200.0
_N_BANDS = 6
_QUANTILE = 0.02


def _band_ranges():
    freq = np.linspace(0.0, _SAMPLE_RATE * 0.5, 1 + _N_FFT // 2)
    octa = np.zeros(_N_BANDS + 2)
    octa[1:] = _FMIN * 2.0 ** np.arange(0, _N_BANDS + 1, dtype=np.float32)
    bands = []
    for k in range(_N_BANDS + 1):
        f_low, f_high = octa[k], octa[k + 1]
        mask = (freq >= f_low) & (freq <= f_high)
        idx = np.flatnonzero(mask)
        if k > 0:
            mask[idx[0] - 1] = True
        if k == _N_BANDS:
            mask[idx[-1] + 1:] = True
        sub_idx = np.flatnonzero(mask)
        if k < _N_BANDS:
            sub_idx = sub_idx[:-1]
        q = int(max(np.rint(_QUANTILE * mask.sum()), 1))
        assert np.all(np.diff(sub_idx) == 1)
        bands.append((int(sub_idx[0]), int(len(sub_idx)), q))
    return bands


_BANDS = _band_ranges()
_NB = len(_BANDS)

def _oe_merge(lo, hi, r):
    step = r * 2
    if step < hi - lo:
        yield from _oe_merge(lo, hi, step)
        yield from _oe_merge(lo + r, hi, step)
        for i in range(lo + r, hi - r, step):
            yield (i, i + r)
    else:
        yield (lo, lo + r)


def _oe_sort_range(lo, hi):
    if hi - lo >= 1:
        mid = lo + (hi - lo) // 2
        yield from _oe_sort_range(lo, mid)
        yield from _oe_sort_range(mid + 1, hi)
        yield from _oe_merge(lo, hi, 1)


_NET16 = tuple(_oe_sort_range(0, 15))


def _pruned_merge_net(q, pad=16):
    inf = object()
    npos = 2 * pad
    pos = [inf] * npos
    for k in range(q):
        pos[k] = k
        pos[pad + k] = q + k
    emitted = []
    for i, j in _oe_merge(0, npos - 1, 1):
        a, b = pos[i], pos[j]
        if b is inf:
            continue
        if a is inf:
            pos[i], pos[j] = b, inf
            continue
        emitted.append((a, b))
    outs = pos[:q]
    needed = set(outs)
    kept = []
    for ij in reversed(emitted):
        if ij[0] in needed or ij[1] in needed:
            kept.append(ij)
            needed.update(ij)
    assert outs == list(range(q))
    return tuple(reversed(kept))


_MERGE_NETS = {q: _pruned_merge_net(q) for q in (6, 9)}
_NET_BANDS = (5, 6)


def _apply_net(v, net, desc):
    for i, j in net:
        a = jnp.minimum(v[i], v[j])
        b = jnp.maximum(v[i], v[j])
        v[i], v[j] = (b, a) if desc else (a, b)


_B, _F, _T = 16, 1025, 2048
_L = 16
_C = 128
_NCHUNK = _T // _C
_NW = 32
_PER_W = _B * _NCHUNK // _NW

_CHUNKS = ((0, 304, (0, 1, 2, 3, 4)), (296, 304, (5,)), (592, 433, (6,)))
_BUF_ROWS = max(c[1] for c in _CHUNKS)


def _sc_body(spec, peak, valley, buf, pk, vl):
    wid = lax.axis_index("s") * 2 + lax.axis_index("c")

    def item(i, carry):
        it = wid * _PER_W + i
        b = it // _NCHUNK
        t0 = (it % _NCHUNK) * _C
        for alo, plen, bids in _CHUNKS:
            pltpu.sync_copy(spec.at[b, pl.ds(alo, plen), pl.ds(t0, _C)],
                            buf.at[pl.ds(0, plen)])
            for g in range(_C // _L):
                sl = pl.ds(g * _L, _L)
                for bi in bids:
                    lo, ln, q = _BANDS[bi]
                    off = lo - alo
                    inf = jnp.full((_L,), jnp.inf, jnp.float32)
                    if bi in _NET_BANDS:
                        net_m = _MERGE_NETS[q]
                        nfull = ln // 16
                        rem = ln - nfull * 16
                        init = ((inf,) * q, (-inf,) * q)

                        def chunk(c, st, off=off, q=q, sl=sl, net_m=net_m):
                            base = off + c * 16
                            s = [buf[base + k, sl] for k in range(16)]
                            _apply_net(s, _NET16, False)
                            vb = list(st[0]) + s[:q]
                            _apply_net(vb, net_m, False)
                            vt = list(st[1]) + [s[15 - k] for k in range(q)]
                            _apply_net(vt, net_m, True)
                            return tuple(vb[:q]), tuple(vt[:q])

                        bot, top = lax.fori_loop(0, nfull, chunk, init)
                        if rem:
                            base = off + nfull * 16
                            s = [buf[base + k, sl] for k in range(rem)]
                            s += [inf] * (16 - rem)
                            _apply_net(s, _NET16, False)
                            vb = list(bot) + s[:q]
                            _apply_net(vb, net_m, False)
                            vt = list(top) + [s[rem - 1 - k] for k in range(q)]
                            _apply_net(vt, net_m, True)
                            bot, top = tuple(vb[:q]), tuple(vt[:q])
                    else:
                        init = ((inf,) * q, (-inf,) * q)

                        def row(r, st, off=off, q=q, sl=sl):
                            bot, top = st
                            v = buf[off + r, sl]
                            w = v
                            nb, nt = [], []
                            for j in range(q):
                                nb.append(jnp.minimum(bot[j], v))
                                v = jnp.maximum(bot[j], v)
                            for j in range(q):
                                nt.append(jnp.maximum(top[j], w))
                                w = jnp.minimum(top[j], w)
                            return tuple(nb), tuple(nt)

                        bot, top = lax.fori_loop(0, ln, row, init)
                    sb, st_ = bot[0], top[0]
                    for j in range(1, q):
                        sb = sb + bot[j]
                        st_ = st_ + top[j]
                    r = jnp.float32(1.0 / q)
                    vl[bi, sl] = sb * r
                    pk[bi, sl] = st_ * r
        pltpu.sync_copy(pk, peak.at[b, :, pl.ds(t0, _C)])
        pltpu.sync_copy(vl, valley.at[b, :, pl.ds(t0, _C)])
        return carry

    lax.fori_loop(0, _PER_W, item, 0)


@functools.lru_cache(maxsize=1)
def _sc_extremes():
    return pl.kernel(
        _sc_body,
        out_type=[jax.ShapeDtypeStruct((_B, _NB, _T), jnp.float32),
                  jax.ShapeDtypeStruct((_B, _NB, _T), jnp.float32)],
        mesh=plsc.VectorSubcoreMesh(core_axis_name="c", subcore_axis_name="s",
                                    num_cores=2, num_subcores=16),
        scratch_types=[pltpu.VMEM((_BUF_ROWS, _C), jnp.float32),
                       pltpu.VMEM((_NB, _C), jnp.float32),
                       pltpu.VMEM((_NB, _C), jnp.float32)],
    )


def _db_body(pk_ref, vl_ref, o_ref):
    pk = pk_ref[0]
    vb = vl_ref[0]
    lp = 10.0 * jnp.log10(jnp.maximum(pk, 1e-10))
    lv = 10.0 * jnp.log10(jnp.maximum(vb, 1e-10))
    lp = jnp.maximum(lp, jnp.max(lp) - 80.0)
    lv = jnp.maximum(lv, jnp.max(lv) - 80.0)
    o_ref[0] = lp - lv


def kernel(spectrogram):
    peak, valley = _sc_extremes()(spectrogram)
    return pl.pallas_call(
        _db_body,
        grid=(_B,),
        in_specs=[pl.BlockSpec((1, _NB, _T), lambda b: (b, 0, 0))] * 2,
        out_specs=pl.BlockSpec((1, _NB, _T), lambda b: (b, 0, 0)),
        out_shape=jax.ShapeDtypeStruct((_B, _NB, _T), jnp.float32),
    )(peak, valley)


# speedup vs baseline: 78.7485x; 1.2849x over previous
import functools

import numpy as np
import jax
import jax.numpy as jnp
from jax import lax
from jax.experimental import pallas as pl
from jax.experimental.pallas import tpu as pltpu
from jax.experimental.pallas import tpu_sc as plsc

_SAMPLE_RATE = 22050
_N_FFT = 2048
_FMIN = 200.0
_N_BANDS = 6
_QUANTILE = 0.02


def _band_ranges():
    freq = np.linspace(0.0, _SAMPLE_RATE * 0.5, 1 + _N_FFT // 2)
    octa = np.zeros(_N_BANDS + 2)
    octa[1:] = _FMIN * 2.0 ** np.arange(0, _N_BANDS + 1, dtype=np.float32)
    bands = []
    for k in range(_N_BANDS + 1):
        f_low, f_high = octa[k], octa[k + 1]
        mask = (freq >= f_low) & (freq <= f_high)
        idx = np.flatnonzero(mask)
        if k > 0:
            mask[idx[0] - 1] = True
        if k == _N_BANDS:
            mask[idx[-1] + 1:] = True
        sub_idx = np.flatnonzero(mask)
        if k < _N_BANDS:
            sub_idx = sub_idx[:-1]
        q = int(max(np.rint(_QUANTILE * mask.sum()), 1))
        assert np.all(np.diff(sub_idx) == 1)
        bands.append((int(sub_idx[0]), int(len(sub_idx)), q))
    return bands


_BANDS = _band_ranges()
_NB = len(_BANDS)

def _oe_merge(lo, hi, r):
    step = r * 2
    if step < hi - lo:
        yield from _oe_merge(lo, hi, step)
        yield from _oe_merge(lo + r, hi, step)
        for i in range(lo + r, hi - r, step):
            yield (i, i + r)
    else:
        yield (lo, lo + r)


def _oe_sort_range(lo, hi):
    if hi - lo >= 1:
        mid = lo + (hi - lo) // 2
        yield from _oe_sort_range(lo, mid)
        yield from _oe_sort_range(mid + 1, hi)
        yield from _oe_merge(lo, hi, 1)


_NET16 = tuple(_oe_sort_range(0, 15))


def _pruned_merge_net(q, pad=16):
    inf = object()
    npos = 2 * pad
    pos = [inf] * npos
    for k in range(q):
        pos[k] = k
        pos[pad + k] = q + k
    emitted = []
    for i, j in _oe_merge(0, npos - 1, 1):
        a, b = pos[i], pos[j]
        if b is inf:
            continue
        if a is inf:
            pos[i], pos[j] = b, inf
            continue
        emitted.append((a, b))
    outs = pos[:q]
    needed = set(outs)
    kept = []
    for ij in reversed(emitted):
        if ij[0] in needed or ij[1] in needed:
            kept.append(ij)
            needed.update(ij)
    assert outs == list(range(q))
    return tuple(reversed(kept))


_MERGE_NETS = {q: _pruned_merge_net(q) for q in (2, 3, 6, 9)}


def _folded_sort16(n_real, outs):
    inf = object()
    pos = list(range(n_real)) + [inf] * (16 - n_real)
    emitted = []
    for i, j in _NET16:
        a, b = pos[i], pos[j]
        if b is inf:
            continue
        if a is inf:
            pos[i], pos[j] = b, inf
            continue
        emitted.append((a, b))
    out_ids = [pos[o] for o in outs]
    assert all(o is not inf for o in out_ids)
    needed = set(out_ids)
    kept = []
    for ij in reversed(emitted):
        if ij[0] in needed or ij[1] in needed:
            kept.append(ij)
            needed.update(ij)
    return tuple(reversed(kept)), tuple(out_ids)


def _sort_net(n_real, q):
    outs = list(range(q)) + [n_real - 1 - k for k in range(q)]
    return _folded_sort16(n_real, outs)


_SORT_NETS = {}
for _q, _rem in ((2, 10), (3, 5), (6, 9), (9, 15)):
    _SORT_NETS[(16, _q)] = _sort_net(16, _q)
    _SORT_NETS[(_rem, _q)] = _sort_net(_rem, _q)


def _apply_net(v, net, desc):
    for i, j in net:
        a = jnp.minimum(v[i], v[j])
        b = jnp.maximum(v[i], v[j])
        v[i], v[j] = (b, a) if desc else (a, b)


def _chunk_step(loads, R, T, q, snet):
    net, ids = snet
    s = list(loads)
    _apply_net(s, net, False)
    vb = list(R) + [s[i] for i in ids[:q]]
    _apply_net(vb, _MERGE_NETS[q], False)
    vt = list(T) + [s[i] for i in ids[q:]]
    _apply_net(vt, _MERGE_NETS[q], True)
    return tuple(vb[:q]), tuple(vt[:q])


def _net_band(buf, sl, q, off, nfull, rem, init):
    snet_full = _SORT_NETS[(16, q)]

    def chunk(c, st):
        base = off + c * 16
        loads = [buf[base + k, sl] for k in range(16)]
        return _chunk_step(loads, st[0], st[1], q, snet_full)

    R, T = lax.fori_loop(0, nfull, chunk, init)
    if rem:
        base = off + nfull * 16
        loads = [buf[base + k, sl] for k in range(rem)]
        R, T = _chunk_step(loads, R, T, q, _SORT_NETS[(rem, q)])
    return R, T


_B, _F, _T = 16, 1025, 2048
_L = 16
_C = 128
_NCHUNK = _T // _C
_NW = 32
_PER_W = _B * _NCHUNK // _NW

_K = ((0, 304), (296, 304), (592, 216), (800, 225))
_BUF_ROWS = max(r for _, r in _K)
_ITEMS = _B * _NCHUNK


def _sc_body(spec, peak, valley, buf0, buf1, pk, vl, rs, ts, sem0, sem1):
    wid = lax.axis_index("s") * 2 + lax.axis_index("c")

    def dma(kidx, it, buf, sem):
        alo, plen = _K[kidx]
        b = it // _NCHUNK
        t0 = (it % _NCHUNK) * _C
        return pltpu.make_async_copy(
            spec.at[b, pl.ds(alo, plen), pl.ds(t0, _C)],
            buf.at[pl.ds(0, plen)], sem)

    dma(0, wid * _PER_W, buf0, sem0).start()

    def item(i, carry):
        it = wid * _PER_W + i
        b = it // _NCHUNK
        t0 = (it % _NCHUNK) * _C
        itn = jnp.minimum(it + 1, _ITEMS - 1)
        inf = jnp.full((_L,), jnp.inf, jnp.float32)
        ninf = -inf

        dma(0, it, buf0, sem0).wait()
        dma(1, it, buf1, sem1).start()
        for g in range(_C // _L):
            sl = pl.ds(g * _L, _L)
            for bi in (0, 1, 2):
                lo, ln, _q = _BANDS[bi]

                def row(r, st, lo=lo, sl=sl):
                    v = buf0[lo + r, sl]
                    return jnp.minimum(st[0], v), jnp.maximum(st[1], v)

                mn, mx = lax.fori_loop(0, ln, row, (inf, ninf))
                vl[bi, sl] = mn
                pk[bi, sl] = mx
            for bi in (3, 4):
                lo, ln, q = _BANDS[bi]
                R, T = _net_band(buf0, sl, q, lo, ln // 16, ln % 16,
                                 ((inf,) * q, (ninf,) * q))
                _emit(pk, vl, bi, sl, q, R, T)

        dma(1, it, buf1, sem1).wait()
        dma(2, it, buf0, sem0).start()
        for g in range(_C // _L):
            sl = pl.ds(g * _L, _L)
            R, T = _net_band(buf1, sl, 6, 1, 18, 9,
                             ((inf,) * 6, (ninf,) * 6))
            _emit(pk, vl, 5, sl, 6, R, T)

        dma(2, it, buf0, sem0).wait()
        dma(3, it, buf1, sem1).start()
        for g in range(_C // _L):
            sl = pl.ds(g * _L, _L)
            R, T = _net_band(buf0, sl, 9, 2, 13, 0,
                             ((inf,) * 9, (ninf,) * 9))
            for j in range(9):
                rs[j, sl] = R[j]
                ts[j, sl] = T[j]

        dma(3, it, buf1, sem1).wait()
        dma(0, itn, buf0, sem0).start()
        for g in range(_C // _L):
            sl = pl.ds(g * _L, _L)
            R = tuple(rs[j, sl] for j in range(9))
            T = tuple(ts[j, sl] for j in range(9))
            R, T = _net_band(buf1, sl, 9, 2, 13, 15, (R, T))
            _emit(pk, vl, 6, sl, 9, R, T)

        pltpu.sync_copy(pk, peak.at[b, :, pl.ds(t0, _C)])
        pltpu.sync_copy(vl, valley.at[b, :, pl.ds(t0, _C)])
        return carry

    lax.fori_loop(0, _PER_W, item, 0)
    dma(0, wid * _PER_W, buf0, sem0).wait()


def _emit(pk, vl, bi, sl, q, R, T):
    sb, st_ = R[0], T[0]
    for j in range(1, q):
        sb = sb + R[j]
        st_ = st_ + T[j]
    r = jnp.float32(1.0 / q)
    vl[bi, sl] = sb * r
    pk[bi, sl] = st_ * r


@functools.lru_cache(maxsize=1)
def _sc_extremes():
    return pl.kernel(
        _sc_body,
        out_type=[jax.ShapeDtypeStruct((_B, _NB, _T), jnp.float32),
                  jax.ShapeDtypeStruct((_B, _NB, _T), jnp.float32)],
        mesh=plsc.VectorSubcoreMesh(core_axis_name="c", subcore_axis_name="s",
                                    num_cores=2, num_subcores=16),
        scratch_types=[pltpu.VMEM((_BUF_ROWS, _C), jnp.float32),
                       pltpu.VMEM((_BUF_ROWS, _C), jnp.float32),
                       pltpu.VMEM((_NB, _C), jnp.float32),
                       pltpu.VMEM((_NB, _C), jnp.float32),
                       pltpu.VMEM((9, _C), jnp.float32),
                       pltpu.VMEM((9, _C), jnp.float32),
                       pltpu.SemaphoreType.DMA,
                       pltpu.SemaphoreType.DMA],
    )


def _db_body(pk_ref, vl_ref, o_ref):
    pk = pk_ref[0]
    vb = vl_ref[0]
    lp = 10.0 * jnp.log10(jnp.maximum(pk, 1e-10))
    lv = 10.0 * jnp.log10(jnp.maximum(vb, 1e-10))
    lp = jnp.maximum(lp, jnp.max(lp) - 80.0)
    lv = jnp.maximum(lv, jnp.max(lv) - 80.0)
    o_ref[0] = lp - lv


def kernel(spectrogram):
    peak, valley = _sc_extremes()(spectrogram)
    return pl.pallas_call(
        _db_body,
        grid=(_B,),
        in_specs=[pl.BlockSpec((1, _NB, _T), lambda b: (b, 0, 0))] * 2,
        out_specs=pl.BlockSpec((1, _NB, _T), lambda b: (b, 0, 0)),
        out_shape=jax.ShapeDtypeStruct((_B, _NB, _T), jnp.float32),
    )(peak, valley)
